# Initial kernel scaffold; baseline (speedup 1.0000x reference)
import jax
import jax.numpy as jnp
from jax.experimental import pallas as pl


def kernel(x, edge_index, edge_attr, candidate_idxs, batch, params):
    raise NotImplementedError("write your pallas kernel here")


# speedup vs baseline: 1.6364x; 1.6364x over previous
import functools

import jax
import jax.numpy as jnp
from jax import lax
from jax.experimental import pallas as pl
from jax.experimental.pallas import tpu as pltpu
from jax.experimental.pallas import tpu_sc as plsc

N = 50000
E = 800000
EMB = 32
ED = 4
NL = 4
NC = 2048
NG = 512

NCORE = 2
NSUB = 16
NW = NCORE * NSUB

EPAD = 819200
EB = 4096
NBLK = 2000
ROWS_PER_TILE = N // NSUB
ZCH = 125

def _wid():
    return lax.axis_index("s") * NCORE + lax.axis_index("c")


@functools.cache
def _make_sc_gather():
    mesh = plsc.VectorSubcoreMesh(core_axis_name="c", subcore_axis_name="s")

    @functools.partial(
        pl.kernel, mesh=mesh,
        compiler_params=pltpu.CompilerParams(use_tc_tiling_on_sc=False),
        out_type=jax.ShapeDtypeStruct((2 * EPAD, EMB), jnp.float32),
        scratch_types=[
            pltpu.VMEM((1280,), jnp.int32),
            pltpu.VMEM((1280, EMB), jnp.float32),
        ],
    )
    def sc_gather(table_hbm, idx_hbm, out_hbm, idx_v, rows_v):
        per_w = (2 * EPAD) // NW
        base = _wid() * per_w

        def outer(o, carry):
            off = base + o * 1280
            pltpu.sync_copy(idx_hbm.at[pl.ds(off, 1280)], idx_v)

            def inner(j, c2):
                sl = pl.ds(j * 128, 128)
                pltpu.sync_copy(table_hbm.at[idx_v.at[sl]], rows_v.at[sl])
                return c2

            lax.fori_loop(0, 10, inner, 0)
            pltpu.sync_copy(rows_v, out_hbm.at[pl.ds(off, 1280)])
            return carry

        lax.fori_loop(0, per_w // 1280, outer, 0)

    return sc_gather


def _sc_gather(table, idx2):
    return _make_sc_gather()(table, idx2)


@functools.cache
def _make_sc_scatter_add():
    mesh = plsc.VectorSubcoreMesh(core_axis_name="c", subcore_axis_name="s")

    @functools.partial(
        pl.kernel, mesh=mesh,
        compiler_params=pltpu.CompilerParams(use_tc_tiling_on_sc=False),
        out_type=jax.ShapeDtypeStruct((N, EMB), jnp.float32),
        scratch_types=[
            pltpu.VMEM((128,), jnp.int32),
            pltpu.VMEM((2560, 16), jnp.float32),
            pltpu.VMEM((ZCH, 16), jnp.float32),
            pltpu.VMEM_SHARED((N, 16), jnp.float32),
        ],
    )
    def sc_scatter_add(z_hbm, dst_hbm, zero_hbm, out_hbm,
                       idx_v, rows_v, bounce_v, acc_sh):
        cid = lax.axis_index("c")
        sid = lax.axis_index("s")
        rb = sid * ROWS_PER_TILE
        colbase = cid * 16

        pltpu.sync_copy(zero_hbm, bounce_v)

        def zinit(k, c):
            pltpu.sync_copy(bounce_v, acc_sh.at[pl.ds(rb + k * ZCH, ZCH)])
            return c

        lax.fori_loop(0, ROWS_PER_TILE // ZCH, zinit, 0)
        plsc.subcore_barrier()

        per_t = EPAD // NSUB
        base = sid * per_t

        def outer(o, carry):
            off = base + o * 2560
            pltpu.sync_copy(z_hbm.at[pl.ds(off, 2560), pl.ds(colbase, 16)],
                            rows_v)

            def inner(j, c2):
                pltpu.sync_copy(dst_hbm.at[pl.ds(off + j * 128, 128)], idx_v)
                pltpu.sync_copy(rows_v.at[pl.ds(j * 128, 128)],
                                acc_sh.at[idx_v], add=True)
                return c2

            lax.fori_loop(0, 20, inner, 0)
            return carry

        lax.fori_loop(0, per_t // 2560, outer, 0)
        plsc.subcore_barrier()

        def wb(k, c):
            sl = pl.ds(rb + k * ZCH, ZCH)
            pltpu.sync_copy(acc_sh.at[sl], bounce_v)
            pltpu.sync_copy(bounce_v,
                            out_hbm.at[sl, pl.ds(colbase, 16)])
            return c

        lax.fori_loop(0, ROWS_PER_TILE // ZCH, wb, 0)

    return sc_scatter_add


def _sc_scatter_add(z, dst_p, zero_rows):
    return _make_sc_scatter_add()(z, dst_p, zero_rows)


@functools.cache
def _make_sc_readout_gather():
    mesh = plsc.VectorSubcoreMesh(core_axis_name="c", subcore_axis_name="s")

    @functools.partial(
        pl.kernel, mesh=mesh,
        compiler_params=pltpu.CompilerParams(use_tc_tiling_on_sc=False),
        out_type=(jax.ShapeDtypeStruct((NC, EMB), jnp.float32),
                  jax.ShapeDtypeStruct((NC, 16), jnp.int32)),
        scratch_types=[
            pltpu.VMEM((64,), jnp.int32),
            pltpu.VMEM((64, EMB), jnp.float32),
            pltpu.VMEM((64, 16), jnp.int32),
        ],
    )
    def sc_readout_gather(h_hbm, batch2_hbm, cand_hbm, hc_hbm, seg_hbm,
                          idx_v, hrows_v, brows_v):
        per_w = NC // NW
        off = _wid() * per_w
        pltpu.sync_copy(cand_hbm.at[pl.ds(off, per_w)], idx_v)
        pltpu.sync_copy(h_hbm.at[idx_v], hrows_v)
        pltpu.sync_copy(hrows_v, hc_hbm.at[pl.ds(off, per_w)])
        pltpu.sync_copy(batch2_hbm.at[idx_v], brows_v)
        pltpu.sync_copy(brows_v, seg_hbm.at[pl.ds(off, per_w)])

    return sc_readout_gather


def _sc_readout_gather(h, batch2, cand):
    return _make_sc_readout_gather()(h, batch2, cand)


def _edge_mask(i, blk):
    eids = i * blk + lax.broadcasted_iota(jnp.int32, (blk, 1), 0)
    return eids < E


def _stats1_body(gu_ref, gv_ref, attr_ref, wa_ref, vec_ref, g_ref, sums_ref):
    i = pl.program_id(0)
    b1 = vec_ref[0:1, :]
    h1 = gu_ref[0] + gv_ref[0] + attr_ref[...] @ wa_ref[...] + b1
    g_ref[...] = h1
    m = _edge_mask(i, EB)
    h1m = jnp.where(m, h1, 0.0)
    s = jnp.sum(h1m, axis=0, keepdims=True)
    s2 = jnp.sum(h1m * h1m, axis=0, keepdims=True)
    upd = jnp.concatenate([s, s2, jnp.zeros((6, EMB), jnp.float32)], axis=0)

    @pl.when(i == 0)
    def _():
        sums_ref[...] = jnp.zeros((8, EMB), jnp.float32)

    sums_ref[...] += upd


def _tc_stats1(gg, attr_p, wa, vec):
    grid = EPAD // EB
    return pl.pallas_call(
        _stats1_body,
        grid=(grid,),
        in_specs=[
            pl.BlockSpec((1, EB, EMB), lambda i: (0, i, 0)),
            pl.BlockSpec((1, EB, EMB), lambda i: (1, i, 0)),
            pl.BlockSpec((EB, ED), lambda i: (i, 0)),
            pl.BlockSpec((ED, EMB), lambda i: (0, 0)),
            pl.BlockSpec((8, EMB), lambda i: (0, 0)),
        ],
        out_specs=[
            pl.BlockSpec((EB, EMB), lambda i: (i, 0)),
            pl.BlockSpec((8, EMB), lambda i: (0, 0)),
        ],
        out_shape=[
            jax.ShapeDtypeStruct((EPAD, EMB), jnp.float32),
            jax.ShapeDtypeStruct((8, EMB), jnp.float32),
        ],
    )(gg, gg, attr_p, wa, vec)


def _stats2_body(g_ref, w2_ref, vec_ref, sums_ref):
    i = pl.program_id(0)
    s1, t1, b2 = vec_ref[0:1, :], vec_ref[1:2, :], vec_ref[2:3, :]
    r1 = jax.nn.relu(s1 * g_ref[...] + t1)
    m = _edge_mask(i, EB)
    r1 = jnp.where(m, r1, 0.0)
    h2 = jnp.where(m, r1 @ w2_ref[...] + b2, 0.0)
    s = jnp.sum(h2, axis=0, keepdims=True)
    s2 = jnp.sum(h2 * h2, axis=0, keepdims=True)
    upd = jnp.concatenate([s, s2, jnp.zeros((6, EMB), jnp.float32)], axis=0)

    @pl.when(i == 0)
    def _():
        sums_ref[...] = jnp.zeros((8, EMB), jnp.float32)

    sums_ref[...] += upd


def _tc_stats2(g, w2, vec):
    grid = EPAD // EB
    return pl.pallas_call(
        _stats2_body,
        grid=(grid,),
        in_specs=[
            pl.BlockSpec((EB, EMB), lambda i: (i, 0)),
            pl.BlockSpec((EMB, EMB), lambda i: (0, 0)),
            pl.BlockSpec((8, EMB), lambda i: (0, 0)),
        ],
        out_specs=pl.BlockSpec((8, EMB), lambda i: (0, 0)),
        out_shape=jax.ShapeDtypeStruct((8, EMB), jnp.float32),
    )(g, w2, vec)


def _passz_body(g_ref, w2_ref, vec_ref, z_ref):
    i = pl.program_id(0)
    s1, t1, b2 = vec_ref[0:1, :], vec_ref[1:2, :], vec_ref[2:3, :]
    s2, t2 = vec_ref[3:4, :], vec_ref[4:5, :]
    r1 = jax.nn.relu(s1 * g_ref[...] + t1)
    h2 = r1 @ w2_ref[...] + b2
    z = jax.nn.relu(s2 * h2 + t2)
    z_ref[...] = jnp.where(_edge_mask(i, EB), z, 0.0)


def _tc_passz(g, w2, vec):
    grid = EPAD // EB
    return pl.pallas_call(
        _passz_body,
        grid=(grid,),
        in_specs=[
            pl.BlockSpec((EB, EMB), lambda i: (i, 0)),
            pl.BlockSpec((EMB, EMB), lambda i: (0, 0)),
            pl.BlockSpec((8, EMB), lambda i: (0, 0)),
        ],
        out_specs=pl.BlockSpec((EB, EMB), lambda i: (i, 0)),
        out_shape=jax.ShapeDtypeStruct((EPAD, EMB), jnp.float32),
    )(g, w2, vec)


def _node_body(h_ref, agg_ref, wd_ref, ws_ref, hn_ref, uv_ref):
    hn = h_ref[...] + agg_ref[...]
    hn_ref[...] = hn
    uv_ref[0] = hn @ wd_ref[...]
    uv_ref[1] = hn @ ws_ref[...]


def _tc_node(h, agg2, wd, ws):
    grid = N // NBLK
    return pl.pallas_call(
        _node_body,
        grid=(grid,),
        in_specs=[
            pl.BlockSpec((NBLK, EMB), lambda i: (i, 0)),
            pl.BlockSpec((NBLK, EMB), lambda i: (i, 0)),
            pl.BlockSpec((EMB, EMB), lambda i: (0, 0)),
            pl.BlockSpec((EMB, EMB), lambda i: (0, 0)),
        ],
        out_specs=[
            pl.BlockSpec((NBLK, EMB), lambda i: (i, 0)),
            pl.BlockSpec((2, NBLK, EMB), lambda i: (0, i, 0)),
        ],
        out_shape=[
            jax.ShapeDtypeStruct((N, EMB), jnp.float32),
            jax.ShapeDtypeStruct((2, N, EMB), jnp.float32),
        ],
    )(h, agg2, wd, ws)


def _prologue_body(x_ref, win_ref, bin_ref, wd_ref, ws_ref, h_ref, uv_ref):
    h = x_ref[...] @ win_ref[...] + bin_ref[0:1, :]
    h_ref[...] = h
    uv_ref[0] = h @ wd_ref[...]
    uv_ref[1] = h @ ws_ref[...]


def _tc_prologue(x, win, binv, wd, ws):
    grid = N // NBLK
    return pl.pallas_call(
        _prologue_body,
        grid=(grid,),
        in_specs=[
            pl.BlockSpec((NBLK, 2), lambda i: (i, 0)),
            pl.BlockSpec((2, EMB), lambda i: (0, 0)),
            pl.BlockSpec((8, EMB), lambda i: (0, 0)),
            pl.BlockSpec((EMB, EMB), lambda i: (0, 0)),
            pl.BlockSpec((EMB, EMB), lambda i: (0, 0)),
        ],
        out_specs=[
            pl.BlockSpec((NBLK, EMB), lambda i: (i, 0)),
            pl.BlockSpec((2, NBLK, EMB), lambda i: (0, i, 0)),
        ],
        out_shape=[
            jax.ShapeDtypeStruct((N, EMB), jnp.float32),
            jax.ShapeDtypeStruct((2, N, EMB), jnp.float32),
        ],
    )(x, win, binv, wd, ws)


def _readout_body(hc_ref, seg_ref, wout_ref, bout_ref, out_ref):
    logits = hc_ref[...] @ wout_ref[...] + bout_ref[0, 0]
    seg = seg_ref[...]
    gids = lax.broadcasted_iota(jnp.int32, (NC, NG), 1)
    mask = seg == gids
    neg = jnp.float32(-1e30)
    mx = jnp.max(jnp.where(mask, logits, neg), axis=0, keepdims=True)
    mxg = jnp.sum(jnp.where(mask, mx, 0.0), axis=1, keepdims=True)
    shifted = logits - mxg
    ex = jnp.exp(shifted)
    ss = jnp.sum(jnp.where(mask, ex, 0.0), axis=0, keepdims=True)
    lse = jnp.log(ss)
    lseg = jnp.sum(jnp.where(mask, lse, 0.0), axis=1, keepdims=True)
    out_ref[...] = jnp.broadcast_to((shifted - lseg).T, (8, NC))


def _tc_readout(hc, seg, wout, bout):
    return pl.pallas_call(
        _readout_body,
        grid=(1,),
        in_specs=[
            pl.BlockSpec((NC, EMB), lambda i: (0, 0)),
            pl.BlockSpec((NC, 1), lambda i: (0, 0)),
            pl.BlockSpec((EMB, 1), lambda i: (0, 0)),
            pl.BlockSpec((1, 1), lambda i: (0, 0)),
        ],
        out_specs=pl.BlockSpec((8, NC), lambda i: (0, 0)),
        out_shape=jax.ShapeDtypeStruct((8, NC), jnp.float32),
    )(hc, seg, wout, bout)


def _bn_affine(sums, gamma, beta):
    mu = sums[0] / E
    var = sums[1] / E - mu * mu
    s = gamma / jnp.sqrt(var + 1e-5)
    t = beta - mu * s
    return s, t


def _pack_rows(*rows):
    out = jnp.zeros((8, EMB), jnp.float32)
    for r, v in enumerate(rows):
        out = out.at[r].set(v)
    return out


def kernel(x, edge_index, edge_attr, candidate_idxs, batch, params):
    f32 = jnp.float32
    dst = edge_index[1]
    src = edge_index[0]
    pad = EPAD - E
    dst_p = jnp.concatenate([dst, jnp.zeros((pad,), jnp.int32)])
    src_p = jnp.concatenate([src, jnp.zeros((pad,), jnp.int32)])
    idx2 = jnp.concatenate([dst_p, src_p + N])
    attr_p = jnp.concatenate([edge_attr, jnp.zeros((pad, ED), f32)])
    zero_rows = jnp.zeros((ZCH, 16), f32)
    batch2 = jnp.broadcast_to(batch[:, None], (N, 16)).astype(jnp.int32)

    layers = params["layers"]
    wds = [p["W1"][0:EMB] for p in layers]
    wss = [p["W1"][EMB:2 * EMB] for p in layers]
    was = [p["W1"][2 * EMB:] for p in layers]

    binv = _pack_rows(params["lin_in_b"])
    h, uv = _tc_prologue(x, params["lin_in_W"], binv, wds[0], wss[0])

    for l in range(NL):
        p = layers[l]
        gg = _sc_gather(uv.reshape(2 * N, EMB), idx2)
        gg3 = gg.reshape(2, EPAD, EMB)
        g, sums1 = _tc_stats1(gg3, attr_p, was[l], _pack_rows(p["b1"]))
        s1, t1 = _bn_affine(sums1, p["g1"], p["be1"])
        vec2 = _pack_rows(s1, t1, p["b2"])
        sums2 = _tc_stats2(g, p["W2"], vec2)
        s2, t2 = _bn_affine(sums2, p["g2"], p["be2"])
        vecz = _pack_rows(s1, t1, p["b2"], s2, t2)
        z = _tc_passz(g, p["W2"], vecz)
        agg = _sc_scatter_add(z, dst_p, zero_rows)
        if l + 1 < NL:
            h, uv = _tc_node(h, agg, wds[l + 1], wss[l + 1])
        else:
            h, uv = _tc_node(h, agg, wds[l], wss[l])

    hc, seg16 = _sc_readout_gather(h, batch2, candidate_idxs)
    seg = seg16[:, 0:1]
    out8 = _tc_readout(hc, seg, params["lin_out_W"],
                       params["lin_out_b"].reshape(1, 1))
    return out8[0]


# speedup vs baseline: 2.9646x; 1.8117x over previous
import functools

import jax
import jax.numpy as jnp
from jax import lax
from jax.experimental import pallas as pl
from jax.experimental.pallas import tpu as pltpu
from jax.experimental.pallas import tpu_sc as plsc

N = 50000
E = 800000
EMB = 32
ED = 4
NL = 4
NC = 2048
NG = 512

NCORE = 2
NSUB = 16
NW = NCORE * NSUB

EPAD = 819200
PE = EPAD // 4
PN = N // 4
EBP = 1024
NBP = 500
ROWS_PER_TILE = N // NSUB
ZCH = 125

f32 = jnp.float32


def _wid():
    return lax.axis_index("s") * NCORE + lax.axis_index("c")


@functools.cache
def _make_sc_gather():
    mesh = plsc.VectorSubcoreMesh(core_axis_name="c", subcore_axis_name="s")

    @functools.partial(
        pl.kernel, mesh=mesh,
        compiler_params=pltpu.CompilerParams(use_tc_tiling_on_sc=False),
        out_type=jax.ShapeDtypeStruct((2 * EPAD, EMB), f32),
        scratch_types=[
            pltpu.VMEM((1280,), jnp.int32),
            pltpu.VMEM((1280, EMB), f32),
        ],
    )
    def sc_gather(table_hbm, idx_hbm, out_hbm, idx_v, rows_v):
        per_w = (2 * EPAD) // NW
        base = _wid() * per_w

        def outer(o, carry):
            off = base + o * 1280
            pltpu.sync_copy(idx_hbm.at[pl.ds(off, 1280)], idx_v)

            def inner(j, c2):
                sl = pl.ds(j * 128, 128)
                pltpu.sync_copy(table_hbm.at[idx_v.at[sl]], rows_v.at[sl])
                return c2

            lax.fori_loop(0, 10, inner, 0)
            pltpu.sync_copy(rows_v, out_hbm.at[pl.ds(off, 1280)])
            return carry

        lax.fori_loop(0, per_w // 1280, outer, 0)

    return sc_gather


def _sc_gather(table, idx2):
    return _make_sc_gather()(table, idx2)


@functools.cache
def _make_sc_scatter_add():
    mesh = plsc.VectorSubcoreMesh(core_axis_name="c", subcore_axis_name="s")

    @functools.partial(
        pl.kernel, mesh=mesh,
        compiler_params=pltpu.CompilerParams(use_tc_tiling_on_sc=False),
        out_type=jax.ShapeDtypeStruct((N, EMB), f32),
        scratch_types=[
            pltpu.VMEM((128,), jnp.int32),
            pltpu.VMEM((2560, 16), f32),
            pltpu.VMEM((ZCH, 16), f32),
            pltpu.VMEM_SHARED((N, 16), f32),
        ],
    )
    def sc_scatter_add(z_hbm, dst_hbm, zero_hbm, out_hbm,
                       idx_v, rows_v, bounce_v, acc_sh):
        cid = lax.axis_index("c")
        sid = lax.axis_index("s")
        rb = sid * ROWS_PER_TILE
        colbase = cid * 16

        pltpu.sync_copy(zero_hbm, bounce_v)

        def zinit(k, c):
            pltpu.sync_copy(bounce_v, acc_sh.at[pl.ds(rb + k * ZCH, ZCH)])
            return c

        lax.fori_loop(0, ROWS_PER_TILE // ZCH, zinit, 0)
        plsc.subcore_barrier()

        per_t = EPAD // NSUB
        base = sid * per_t

        def outer(o, carry):
            off = base + o * 2560
            pltpu.sync_copy(z_hbm.at[pl.ds(off, 2560), pl.ds(colbase, 16)],
                            rows_v)

            def inner(j, c2):
                pltpu.sync_copy(dst_hbm.at[pl.ds(off + j * 128, 128)], idx_v)
                pltpu.sync_copy(rows_v.at[pl.ds(j * 128, 128)],
                                acc_sh.at[idx_v], add=True)
                return c2

            lax.fori_loop(0, 20, inner, 0)
            return carry

        lax.fori_loop(0, per_t // 2560, outer, 0)
        plsc.subcore_barrier()

        def wb(k, c):
            sl = pl.ds(rb + k * ZCH, ZCH)
            pltpu.sync_copy(acc_sh.at[sl], bounce_v)
            pltpu.sync_copy(bounce_v, out_hbm.at[sl, pl.ds(colbase, 16)])
            return c

        lax.fori_loop(0, ROWS_PER_TILE // ZCH, wb, 0)

    return sc_scatter_add


def _sc_scatter_add(z, dst_p, zero_rows):
    return _make_sc_scatter_add()(z, dst_p, zero_rows)


@functools.cache
def _make_sc_readout_gather():
    mesh = plsc.VectorSubcoreMesh(core_axis_name="c", subcore_axis_name="s")

    @functools.partial(
        pl.kernel, mesh=mesh,
        compiler_params=pltpu.CompilerParams(use_tc_tiling_on_sc=False),
        out_type=(jax.ShapeDtypeStruct((NC, EMB), f32),
                  jax.ShapeDtypeStruct((NC, 16), jnp.int32)),
        scratch_types=[
            pltpu.VMEM((64,), jnp.int32),
            pltpu.VMEM((64, EMB), f32),
            pltpu.VMEM((64, 16), jnp.int32),
        ],
    )
    def sc_readout_gather(h_hbm, batch2_hbm, cand_hbm, hc_hbm, seg_hbm,
                          idx_v, hrows_v, brows_v):
        per_w = NC // NW
        off = _wid() * per_w
        pltpu.sync_copy(cand_hbm.at[pl.ds(off, per_w)], idx_v)
        pltpu.sync_copy(h_hbm.at[idx_v], hrows_v)
        pltpu.sync_copy(hrows_v, hc_hbm.at[pl.ds(off, per_w)])
        pltpu.sync_copy(batch2_hbm.at[idx_v], brows_v)
        pltpu.sync_copy(brows_v, seg_hbm.at[pl.ds(off, per_w)])

    return sc_readout_gather


def _sc_readout_gather(h, batch2, cand):
    return _make_sc_readout_gather()(h, batch2, cand)


def _edge_mask_packed(i):
    row = lax.broadcasted_iota(jnp.int32, (EBP, 128), 0)
    lane = lax.broadcasted_iota(jnp.int32, (EBP, 128), 1)
    eid = 4 * (i * EBP + row) + lane // EMB
    return eid < E


def _stats1_body(gu_ref, gv_ref, attr_ref, wa_ref, vec_ref, g_ref, sums_ref):
    i = pl.program_id(0)
    b1 = vec_ref[0:1, :]
    h1 = gu_ref[...] + gv_ref[...] + attr_ref[...] @ wa_ref[...] + b1
    g_ref[...] = h1
    m = _edge_mask_packed(i)
    h1m = jnp.where(m, h1, 0.0)
    s = jnp.sum(h1m, axis=0, keepdims=True)
    s2 = jnp.sum(h1m * h1m, axis=0, keepdims=True)
    upd = jnp.concatenate([s, s2, jnp.zeros((6, 128), f32)], axis=0)

    @pl.when(i == 0)
    def _():
        sums_ref[...] = jnp.zeros((8, 128), f32)

    sums_ref[...] += upd


def _tc_stats1(gg_p, attr16, wa_bd, vec):
    grid = PE // EBP
    return pl.pallas_call(
        _stats1_body,
        grid=(grid,),
        in_specs=[
            pl.BlockSpec((EBP, 128), lambda i: (i, 0)),
            pl.BlockSpec((EBP, 128), lambda i: (grid + i, 0)),
            pl.BlockSpec((EBP, 16), lambda i: (i, 0)),
            pl.BlockSpec((16, 128), lambda i: (0, 0)),
            pl.BlockSpec((8, 128), lambda i: (0, 0)),
        ],
        out_specs=[
            pl.BlockSpec((EBP, 128), lambda i: (i, 0)),
            pl.BlockSpec((8, 128), lambda i: (0, 0)),
        ],
        out_shape=[
            jax.ShapeDtypeStruct((PE, 128), f32),
            jax.ShapeDtypeStruct((8, 128), f32),
        ],
    )(gg_p, gg_p, attr16, wa_bd, vec)


def _stats2_body(g_ref, w2_ref, vec_ref, sums_ref):
    i = pl.program_id(0)
    s1, t1, b2 = vec_ref[0:1, :], vec_ref[1:2, :], vec_ref[2:3, :]
    r1 = jax.nn.relu(s1 * g_ref[...] + t1)
    m = _edge_mask_packed(i)
    r1 = jnp.where(m, r1, 0.0)
    h2 = jnp.where(m, r1 @ w2_ref[...] + b2, 0.0)
    s = jnp.sum(h2, axis=0, keepdims=True)
    s2 = jnp.sum(h2 * h2, axis=0, keepdims=True)
    upd = jnp.concatenate([s, s2, jnp.zeros((6, 128), f32)], axis=0)

    @pl.when(i == 0)
    def _():
        sums_ref[...] = jnp.zeros((8, 128), f32)

    sums_ref[...] += upd


def _tc_stats2(g_p, w2_bd, vec):
    grid = PE // EBP
    return pl.pallas_call(
        _stats2_body,
        grid=(grid,),
        in_specs=[
            pl.BlockSpec((EBP, 128), lambda i: (i, 0)),
            pl.BlockSpec((128, 128), lambda i: (0, 0)),
            pl.BlockSpec((8, 128), lambda i: (0, 0)),
        ],
        out_specs=pl.BlockSpec((8, 128), lambda i: (0, 0)),
        out_shape=jax.ShapeDtypeStruct((8, 128), f32),
    )(g_p, w2_bd, vec)


def _passz_body(g_ref, w2_ref, vec_ref, z_ref):
    i = pl.program_id(0)
    s1, t1, b2 = vec_ref[0:1, :], vec_ref[1:2, :], vec_ref[2:3, :]
    s2, t2 = vec_ref[3:4, :], vec_ref[4:5, :]
    r1 = jax.nn.relu(s1 * g_ref[...] + t1)
    h2 = r1 @ w2_ref[...] + b2
    z = jax.nn.relu(s2 * h2 + t2)
    z_ref[...] = jnp.where(_edge_mask_packed(i), z, 0.0)


def _tc_passz(g_p, w2_bd, vec):
    grid = PE // EBP
    return pl.pallas_call(
        _passz_body,
        grid=(grid,),
        in_specs=[
            pl.BlockSpec((EBP, 128), lambda i: (i, 0)),
            pl.BlockSpec((128, 128), lambda i: (0, 0)),
            pl.BlockSpec((8, 128), lambda i: (0, 0)),
        ],
        out_specs=pl.BlockSpec((EBP, 128), lambda i: (i, 0)),
        out_shape=jax.ShapeDtypeStruct((PE, 128), f32),
    )(g_p, w2_bd, vec)


def _node_body(h_ref, agg_ref, wd_ref, ws_ref, hn_ref, uv_ref):
    hn = h_ref[...] + agg_ref[...]
    hn_ref[...] = hn
    uv_ref[0] = hn @ wd_ref[...]
    uv_ref[1] = hn @ ws_ref[...]


def _tc_node(h_p, agg_p, wd_bd, ws_bd):
    return pl.pallas_call(
        _node_body,
        grid=(1,),
        in_specs=[
            pl.BlockSpec((PN, 128), lambda i: (0, 0)),
            pl.BlockSpec((PN, 128), lambda i: (0, 0)),
            pl.BlockSpec((128, 128), lambda i: (0, 0)),
            pl.BlockSpec((128, 128), lambda i: (0, 0)),
        ],
        out_specs=[
            pl.BlockSpec((PN, 128), lambda i: (0, 0)),
            pl.BlockSpec((2, PN, 128), lambda i: (0, 0, 0)),
        ],
        out_shape=[
            jax.ShapeDtypeStruct((PN, 128), f32),
            jax.ShapeDtypeStruct((2, PN, 128), f32),
        ],
    )(h_p, agg_p, wd_bd, ws_bd)


def _prologue_body(x_ref, win_ref, bin_ref, wd_ref, ws_ref, h_ref, uv_ref):
    h = x_ref[...] @ win_ref[...] + bin_ref[0:1, :]
    h_ref[...] = h
    uv_ref[0] = h @ wd_ref[...]
    uv_ref[1] = h @ ws_ref[...]


def _tc_prologue(x_p, win_bd, binv, wd_bd, ws_bd):
    return pl.pallas_call(
        _prologue_body,
        grid=(1,),
        in_specs=[
            pl.BlockSpec((PN, 8), lambda i: (0, 0)),
            pl.BlockSpec((8, 128), lambda i: (0, 0)),
            pl.BlockSpec((8, 128), lambda i: (0, 0)),
            pl.BlockSpec((128, 128), lambda i: (0, 0)),
            pl.BlockSpec((128, 128), lambda i: (0, 0)),
        ],
        out_specs=[
            pl.BlockSpec((PN, 128), lambda i: (0, 0)),
            pl.BlockSpec((2, PN, 128), lambda i: (0, 0, 0)),
        ],
        out_shape=[
            jax.ShapeDtypeStruct((PN, 128), f32),
            jax.ShapeDtypeStruct((2, PN, 128), f32),
        ],
    )(x_p, win_bd, binv, wd_bd, ws_bd)


def _readout_body(hc_ref, seg_ref, wout_ref, bout_ref, out_ref):
    logits = hc_ref[...] @ wout_ref[...] + bout_ref[0, 0]
    seg = seg_ref[...]
    gids = lax.broadcasted_iota(jnp.int32, (NC, NG), 1)
    mask = seg == gids
    neg = jnp.float32(-1e30)
    mx = jnp.max(jnp.where(mask, logits, neg), axis=0, keepdims=True)
    mxg = jnp.sum(jnp.where(mask, mx, 0.0), axis=1, keepdims=True)
    shifted = logits - mxg
    ex = jnp.exp(shifted)
    ss = jnp.sum(jnp.where(mask, ex, 0.0), axis=0, keepdims=True)
    lse = jnp.log(ss)
    lseg = jnp.sum(jnp.where(mask, lse, 0.0), axis=1, keepdims=True)
    out_ref[...] = jnp.broadcast_to((shifted - lseg).T, (8, NC))


def _tc_readout(hc, seg, wout, bout):
    return pl.pallas_call(
        _readout_body,
        grid=(1,),
        in_specs=[
            pl.BlockSpec((NC, EMB), lambda i: (0, 0)),
            pl.BlockSpec((NC, 1), lambda i: (0, 0)),
            pl.BlockSpec((EMB, 1), lambda i: (0, 0)),
            pl.BlockSpec((1, 1), lambda i: (0, 0)),
        ],
        out_specs=pl.BlockSpec((8, NC), lambda i: (0, 0)),
        out_shape=jax.ShapeDtypeStruct((8, NC), f32),
    )(hc, seg, wout, bout)


def _bn_affine(sums, gamma, beta):
    mu = sums[0].reshape(4, EMB).sum(0) / E
    ex2 = sums[1].reshape(4, EMB).sum(0) / E
    var = ex2 - mu * mu
    s = gamma / jnp.sqrt(var + 1e-5)
    t = beta - mu * s
    return s, t


def _pack_rows(*rows):
    out = jnp.zeros((8, 128), f32)
    for r, v in enumerate(rows):
        out = out.at[r].set(jnp.tile(v, 4))
    return out


def _bd(w):
    return jnp.kron(jnp.eye(4, dtype=f32), w)


def kernel(x, edge_index, edge_attr, candidate_idxs, batch, params):
    dst = edge_index[1]
    src = edge_index[0]
    pad = EPAD - E
    dst_p = jnp.concatenate([dst, jnp.zeros((pad,), jnp.int32)])
    src_p = jnp.concatenate([src, jnp.zeros((pad,), jnp.int32)])
    idx2 = jnp.concatenate([dst_p, src_p + N])
    attr16 = jnp.concatenate(
        [edge_attr, jnp.zeros((pad, ED), f32)]).reshape(PE, 4 * ED)
    zero_rows = jnp.zeros((ZCH, 16), f32)

    layers = params["layers"]
    wd_bds = [_bd(p["W1"][0:EMB]) for p in layers]
    ws_bds = [_bd(p["W1"][EMB:2 * EMB]) for p in layers]
    wa_bds = [_bd(p["W1"][2 * EMB:]) for p in layers]
    w2_bds = [_bd(p["W2"]) for p in layers]

    x_p = x.reshape(PN, 8)
    h_p, uv_p = _tc_prologue(x_p, _bd(params["lin_in_W"]),
                             _pack_rows(params["lin_in_b"]),
                             wd_bds[0], ws_bds[0])

    for l in range(NL):
        p = layers[l]
        gg = _sc_gather(uv_p.reshape(2 * N, EMB), idx2)
        gg_p = gg.reshape(2 * PE, 128)
        g_p, sums1 = _tc_stats1(gg_p, attr16, wa_bds[l], _pack_rows(p["b1"]))
        s1, t1 = _bn_affine(sums1, p["g1"], p["be1"])
        sums2 = _tc_stats2(g_p, w2_bds[l], _pack_rows(s1, t1, p["b2"]))
        s2, t2 = _bn_affine(sums2, p["g2"], p["be2"])
        z_p = _tc_passz(g_p, w2_bds[l],
                        _pack_rows(s1, t1, p["b2"], s2, t2))
        agg = _sc_scatter_add(z_p.reshape(EPAD, EMB), dst_p, zero_rows)
        nl_ = min(l + 1, NL - 1)
        h_p, uv_p = _tc_node(h_p, agg.reshape(PN, 128),
                             wd_bds[nl_], ws_bds[nl_])

    batch2 = jnp.repeat(batch.reshape(N // 8, 8), 16, axis=1).reshape(N, 16)
    hc, seg16 = _sc_readout_gather(h_p.reshape(N, EMB), batch2,
                                   candidate_idxs)
    out8 = _tc_readout(hc, seg16[:, 0:1], params["lin_out_W"],
                       params["lin_out_b"].reshape(1, 1))
    return out8[0]
